# Initial kernel scaffold; baseline (speedup 1.0000x reference)
import jax
import jax.numpy as jnp
from jax.experimental import pallas as pl


def kernel(x, edge_index, W, b, bn_gamma, bn_beta, lin_W, lin_b):
    raise NotImplementedError("write your pallas kernel here")


# speedup vs baseline: 5.5989x; 5.5989x over previous
import functools
import math

import jax
import jax.numpy as jnp
from jax import lax
from jax.experimental import pallas as pl
from jax.experimental.pallas import tpu as pltpu
from jax.experimental.pallas import tpu_sc as plsc

N = 10000
E = 320000
D = 128
OUT = 64
L = 5
ALPHA = 0.1
BETA = float(math.log(1.0 / 128.0 + 1.0))
EPS = 1e-5

NC = 2
NS = 16
CHUNK = 128
NCHUNKS = E // CHUNK
CH_PER_CORE = NCHUNKS // NC
CH_BASE = CH_PER_CORE // NS
CH_REM = CH_PER_CORE - CH_BASE * NS
ROWS_PER_TILE = 640
N_PAD = ROWS_PER_TILE * NS
DEGW = 128


def _sc_mesh():
    return plsc.VectorSubcoreMesh(core_axis_name="c", subcore_axis_name="s")


def _edge_body(feat_hbm, src_hbm, dst_hbm, zrow_hbm, out0_hbm, out1_hbm,
               agg, src_v, dst_v, rows_v, sem):
    c = lax.axis_index("c")
    s = lax.axis_index("s")
    rbase = s * ROWS_PER_TILE
    pltpu.sync_copy(zrow_hbm, agg.at[pl.ds(rbase, ROWS_PER_TILE)])
    plsc.subcore_barrier()

    nch = CH_BASE + (s < CH_REM).astype(jnp.int32)

    def body(k, carry):
        chunk = c * CH_PER_CORE + k * NS + s
        base = chunk * CHUNK
        pltpu.sync_copy(src_hbm.at[pl.ds(base, CHUNK)], src_v)
        pltpu.sync_copy(dst_hbm.at[pl.ds(base, CHUNK)], dst_v)
        pltpu.async_copy(feat_hbm.at[src_v], rows_v, sem).wait()
        pltpu.sync_copy(rows_v, agg.at[dst_v], add=True)
        return carry

    lax.fori_loop(0, nch, body, 0)
    plsc.subcore_barrier()

    @pl.when(c == 0)
    def _():
        pltpu.sync_copy(agg.at[pl.ds(rbase, ROWS_PER_TILE)],
                        out0_hbm.at[pl.ds(rbase, ROWS_PER_TILE)])

    @pl.when(c == 1)
    def _():
        pltpu.sync_copy(agg.at[pl.ds(rbase, ROWS_PER_TILE)],
                        out1_hbm.at[pl.ds(rbase, ROWS_PER_TILE)])


_edge_call = functools.partial(
    pl.kernel,
    out_type=(jax.ShapeDtypeStruct((N_PAD, D), jnp.float32),
              jax.ShapeDtypeStruct((N_PAD, D), jnp.float32)),
    scratch_types=[
        pltpu.VMEM_SHARED((N_PAD, D), jnp.float32),
        pltpu.VMEM((CHUNK,), jnp.int32),
        pltpu.VMEM((CHUNK,), jnp.int32),
        pltpu.VMEM((CHUNK, D), jnp.float32),
        pltpu.SemaphoreType.DMA,
    ],
)(_edge_body, mesh=_sc_mesh())


def _deg_body(dst_hbm, zrow_hbm, ones_hbm, out0_hbm, out1_hbm,
              agg, dst_v, ones_v, sem):
    c = lax.axis_index("c")
    s = lax.axis_index("s")
    rbase = s * ROWS_PER_TILE
    pltpu.sync_copy(zrow_hbm, agg.at[pl.ds(rbase, ROWS_PER_TILE)])
    pltpu.sync_copy(ones_hbm, ones_v)
    plsc.subcore_barrier()

    nch = CH_BASE + (s < CH_REM).astype(jnp.int32)

    def body(k, carry):
        chunk = c * CH_PER_CORE + k * NS + s
        base = chunk * CHUNK
        pltpu.sync_copy(dst_hbm.at[pl.ds(base, CHUNK)], dst_v)
        pltpu.sync_copy(ones_v, agg.at[dst_v], add=True)
        return carry

    lax.fori_loop(0, nch, body, 0)
    plsc.subcore_barrier()

    @pl.when(c == 0)
    def _():
        pltpu.sync_copy(agg.at[pl.ds(rbase, ROWS_PER_TILE)],
                        out0_hbm.at[pl.ds(rbase, ROWS_PER_TILE)])

    @pl.when(c == 1)
    def _():
        pltpu.sync_copy(agg.at[pl.ds(rbase, ROWS_PER_TILE)],
                        out1_hbm.at[pl.ds(rbase, ROWS_PER_TILE)])


_deg_call = functools.partial(
    pl.kernel,
    out_type=(jax.ShapeDtypeStruct((N_PAD, DEGW), jnp.float32),
              jax.ShapeDtypeStruct((N_PAD, DEGW), jnp.float32)),
    scratch_types=[
        pltpu.VMEM_SHARED((N_PAD, DEGW), jnp.float32),
        pltpu.VMEM((CHUNK,), jnp.int32),
        pltpu.VMEM((CHUNK, DEGW), jnp.float32),
        pltpu.SemaphoreType.DMA,
    ],
)(_deg_body, mesh=_sc_mesh())


def _prologue_tc(d0_ref, d1_ref, x_ref, norm_ref, feat_ref, pool_ref):
    deg = d0_ref[:, 0:1] + d1_ref[:, 0:1]
    norm = lax.rsqrt(jnp.maximum(deg, 1.0))
    x = x_ref[...]
    norm_ref[...] = norm
    feat_ref[...] = x * norm
    pool_ref[...] = jnp.sum(x, axis=0, keepdims=True)


def _layer_tc(p0_ref, p1_ref, h_ref, norm_ref, w_ref, b_ref, g_ref, be_ref,
              h_out_ref, feat_ref, pool_ref):
    norm = norm_ref[...]
    agg = (p0_ref[...] + p1_ref[...]) * norm
    h = h_ref[...]
    support = (1.0 - ALPHA) * agg + ALPHA * h
    rst = ((1.0 - BETA) * support
           + BETA * jnp.dot(support, w_ref[...],
                            preferred_element_type=jnp.float32)
           + b_ref[...])
    mean = jnp.mean(rst, axis=0, keepdims=True)
    var = jnp.mean((rst - mean) ** 2, axis=0, keepdims=True)
    hn = (rst - mean) * lax.rsqrt(var + EPS)
    hh = jnp.maximum(hn * g_ref[...] + be_ref[...], 0.0)
    h_out_ref[...] = hh
    feat_ref[...] = hh * norm
    pool_ref[...] = jnp.sum(hh, axis=0, keepdims=True)


def _head_tc(pool_ref, lw_ref, lb_ref, out1_ref, out2_ref):
    score = jnp.sum(lb_ref[...], axis=0, keepdims=True)
    for i in range(L + 1):
        p = pool_ref[i:i + 1, :]
        w = lw_ref[i]
        score = score + lax.dot_general(
            p, w, (((1,), (1,)), ((), ())),
            preferred_element_type=jnp.float32)
    m = jnp.max(score, axis=1, keepdims=True)
    lse = m + jnp.log(jnp.sum(jnp.exp(score - m), axis=1, keepdims=True))
    out1_ref[...] = score - lse
    acc = pool_ref[1:2, :]
    for i in range(2, L + 1):
        acc = acc + pool_ref[i:i + 1, :]
    out2_ref[...] = acc * (1.0 / L)


def kernel(x, edge_index, W, b, bn_gamma, bn_beta, lin_W, lin_b):
    src = edge_index[0]
    dst = edge_index[1]
    zrow = jnp.zeros((ROWS_PER_TILE, D), jnp.float32)
    zdeg = jnp.zeros((ROWS_PER_TILE, DEGW), jnp.float32)
    ones_chunk = jnp.ones((CHUNK, DEGW), jnp.float32)

    d0, d1 = _deg_call(dst, zdeg, ones_chunk)
    d0 = d0[:N]
    d1 = d1[:N]

    norm, feat, pool0 = pl.pallas_call(
        _prologue_tc,
        out_shape=(jax.ShapeDtypeStruct((N, 1), jnp.float32),
                   jax.ShapeDtypeStruct((N, D), jnp.float32),
                   jax.ShapeDtypeStruct((1, D), jnp.float32)),
    )(d0, d1, x)

    h = x
    pools = [pool0]
    layer_call = pl.pallas_call(
        _layer_tc,
        out_shape=(jax.ShapeDtypeStruct((N, D), jnp.float32),
                   jax.ShapeDtypeStruct((N, D), jnp.float32),
                   jax.ShapeDtypeStruct((1, D), jnp.float32)),
    )
    for l in range(L):
        p0, p1 = _edge_call(feat, src, dst, zrow)
        p0 = p0[:N]
        p1 = p1[:N]
        h, feat, pool_l = layer_call(
            p0, p1, h, norm, W[l], b[l][None, :],
            bn_gamma[l][None, :], bn_beta[l][None, :])
        pools.append(pool_l)

    pool_all = jnp.concatenate(pools, axis=0)
    out1, out2 = pl.pallas_call(
        _head_tc,
        out_shape=(jax.ShapeDtypeStruct((1, OUT), jnp.float32),
                   jax.ShapeDtypeStruct((1, D), jnp.float32)),
    )(pool_all, lin_W, lin_b)
    return out1, out2
